# baseline (speedup 1.0000x reference)
import functools

import jax
import jax.numpy as jnp
from jax import lax
from jax.experimental import pallas as pl
from jax.experimental.pallas import tpu as pltpu
from jax.experimental.pallas import tpu_sc as plsc

C = 512
CW = C // 2
PAD = 64
V = 3 * PAD
M = 2048

NC, NS, L = 2, 16, 16
NW = NC * NS
TOK = 16 * 2048
TPW = TOK // NW
SUB = 128
NSUB = TPW // SUB


def _prefuse_body(tabs_ref, w1_ref, g_ref):
    for k in range(3):
        t = tabs_ref[k * PAD:(k + 1) * PAD, :]
        w = w1_ref[k * C:(k + 1) * C, :]
        g = jnp.dot(t, w, preferred_element_type=jnp.float32)
        g_ref[k * PAD:(k + 1) * PAD, :] = g.astype(jnp.bfloat16)


def _gather_body(g_hbm, ia_hbm, ir_hbm, ip_hbm, y_hbm,
                 g_vm, y_vm, ia_sm, ir_sm, ip_sm):
    wid = lax.axis_index("s") * NC + lax.axis_index("c")
    base = wid * TPW
    pltpu.sync_copy(g_hbm, g_vm)

    def sub_body(s, _):
        sub_base = base + s * SUB
        pltpu.sync_copy(ia_hbm.at[pl.ds(sub_base, SUB)], ia_sm)
        pltpu.sync_copy(ir_hbm.at[pl.ds(sub_base, SUB)], ir_sm)
        pltpu.sync_copy(ip_hbm.at[pl.ds(sub_base, SUB)], ip_sm)

        def group_body(gi, _):
            va = ia_sm[pl.ds(gi * L, L)]
            vr = ir_sm[pl.ds(gi * L, L)]
            vp = ip_sm[pl.ds(gi * L, L)]
            for l in range(L):
                ta = va[l]
                tr = vr[l]
                tp = vp[l]
                i = gi * L + l
                for j in range(CW // L):
                    col = pl.ds(j * L, L)
                    acc = (plsc.bitcast(g_vm[ta, col], jnp.bfloat16)
                           + plsc.bitcast(g_vm[tr, col], jnp.bfloat16)
                           + plsc.bitcast(g_vm[tp, col], jnp.bfloat16))
                    y_vm[i, col] = plsc.bitcast(acc, jnp.int32)
            return 0

        lax.fori_loop(0, SUB // L, group_body, 0)
        pltpu.sync_copy(y_vm, y_hbm.at[pl.ds(sub_base, SUB)])
        return 0

    lax.fori_loop(0, NSUB, sub_body, 0)


def _mlp_body(y_ref, mask_ref, b1_ref, w2_ref, b2_ref, out_ref):
    y = y_ref[...].astype(jnp.float32) + b1_ref[...]
    h = (y * 0.5 * (1.0 + lax.erf(y * 0.7071067811865476))).astype(jnp.bfloat16)
    out = jnp.dot(h, w2_ref[...], preferred_element_type=jnp.float32)
    out = out + b2_ref[...]
    out_ref[...] = out * mask_ref[0, 0, :][:, None]


_sc_gather = functools.partial(
    pl.kernel,
    out_type=jax.ShapeDtypeStruct((TOK, CW), jnp.int32),
    mesh=plsc.VectorSubcoreMesh(core_axis_name="c", subcore_axis_name="s",
                                num_cores=NC, num_subcores=NS),
    compiler_params=pltpu.CompilerParams(needs_layout_passes=False),
    scratch_types=[
        pltpu.VMEM((V, CW), jnp.int32),
        pltpu.VMEM((SUB, CW), jnp.int32),
        pltpu.VMEM((SUB,), jnp.int32),
        pltpu.VMEM((SUB,), jnp.int32),
        pltpu.VMEM((SUB,), jnp.int32),
    ],
)(_gather_body)


def kernel(atom_type, aa_type, aa_pos, mask, atom_table, residue_table,
           pos_table, W1, b1, W2, b2):
    B, N = atom_type.shape
    T = B * N

    tabs = jnp.zeros((V, C), jnp.float32)
    tabs = tabs.at[0:55].set(atom_table)
    tabs = tabs.at[PAD:PAD + 21].set(residue_table)
    tabs = tabs.at[2 * PAD:2 * PAD + 24].set(pos_table)

    g = pl.pallas_call(
        _prefuse_body,
        out_shape=jax.ShapeDtypeStruct((V, C), jnp.bfloat16),
    )(tabs, W1)

    g_i32 = lax.bitcast_convert_type(g.reshape(V, CW, 2), jnp.int32)

    ia = atom_type.reshape(T).astype(jnp.int32)
    ir = (aa_type.reshape(T) + PAD).astype(jnp.int32)
    ip = (aa_pos.reshape(T) + 2 * PAD).astype(jnp.int32)

    y_i32 = _sc_gather(g_i32, ia, ir, ip)
    y = lax.bitcast_convert_type(y_i32, jnp.bfloat16).reshape(T, C)

    mask_f = mask.reshape(T // M, 1, M).astype(jnp.float32)
    full = lambda shape: pl.BlockSpec(shape, lambda i: (0,) * len(shape))
    out = pl.pallas_call(
        _mlp_body,
        grid=(T // M,),
        in_specs=[pl.BlockSpec((M, C), lambda i: (i, 0)),
                  pl.BlockSpec((1, 1, M), lambda i: (i, 0, 0)),
                  full((1, C)), full((C, C)), full((1, C))],
        out_specs=pl.BlockSpec((M, C), lambda i: (i, 0)),
        out_shape=jax.ShapeDtypeStruct((T, C), jnp.float32),
    )(y, mask_f, b1.reshape(1, C), W2.astype(jnp.bfloat16), b2.reshape(1, C))

    return out.reshape(B, N, C)
